# baseline (speedup 1.0000x reference)
import functools
import math

import jax
import jax.numpy as jnp
from jax.experimental import pallas as pl
from jax.experimental.pallas import tpu as pltpu

BLKQ, BLKK = 64, 32
TOPK_FRAC = 0.1


def _stats_kernel(q_ref, k_ref, v_ref, km_ref, kvsum_ref, ksum_ref, lut_ref,
                  *, mb, nb, topk):
    q = q_ref[0]
    k = k_ref[0]
    v = v_ref[0]
    l, d = q.shape
    km = jnp.mean(k, axis=0, keepdims=True)
    km_ref[0] = km
    pq = jnp.mean(q.reshape(mb, BLKQ, d), axis=1)
    pk = jnp.mean(k.reshape(nb, BLKK, d), axis=1) - km
    ps = jax.lax.dot_general(pq, pk, (((1,), (1,)), ((), ())),
                             precision=jax.lax.Precision.HIGHEST,
                             preferred_element_type=jnp.float32)
    col = jax.lax.broadcasted_iota(jnp.int32, ps.shape, 1)
    for t in range(topk):
        mx = jnp.max(ps, axis=1, keepdims=True)
        idx = jnp.min(jnp.where(ps >= mx, col, nb), axis=1, keepdims=True)
        lut_ref[0, :, t:t + 1] = idx
        ps = jnp.where(col == idx, -jnp.inf, ps)
    kf = jax.nn.softmax(k, axis=-1)
    kvsum_ref[0] = jax.lax.dot_general(kf, v, (((0,), (0,)), ((), ())),
                                       preferred_element_type=jnp.float32)
    ksum_ref[0] = jnp.sum(kf, axis=0, keepdims=True)


def _attn_kernel(lut_ref, q_ref, k_ref, v_ref, kvsum_ref, ksum_ref,
                 w_ref, b_ref, o_ref, kc_scr, vc_scr, *, topk, scale, mg):
    h = pl.program_id(0)
    jg = pl.program_id(1)
    ks = ksum_ref[0]
    kv = kvsum_ref[0]
    w = w_ref[...]
    bb = b_ref[0]
    for g in range(mg):
        m = jg * mg + g
        for t in range(topk):
            idx = lut_ref[h, m, t]
            off = idx * BLKK
            kc_scr[g, t * BLKK:(t + 1) * BLKK, :] = (
                k_ref[0, pl.ds(off, BLKK), :].astype(jnp.bfloat16))
            vc_scr[g, t * BLKK:(t + 1) * BLKK, :] = (
                v_ref[0, pl.ds(off, BLKK), :].astype(jnp.bfloat16))
    for g in range(mg):
        qb = q_ref[0, g * BLKQ:(g + 1) * BLKQ, :]
        qbs = (qb * scale).astype(jnp.bfloat16)
        s = jax.lax.dot_general(qbs, kc_scr[g], (((1,), (1,)), ((), ())),
                                preferred_element_type=jnp.float32)
        s = s - jnp.max(s, axis=1, keepdims=True)
        p = jnp.exp(s)
        rs = jnp.sum(p, axis=1, keepdims=True)
        o_s = jax.lax.dot_general(p.astype(jnp.bfloat16), vc_scr[g],
                                  (((1,), (0,)), ((), ())),
                                  preferred_element_type=jnp.float32) / rs
        qf = jax.nn.softmax(qb, axis=-1)
        denom = jnp.sum(qf * ks, axis=1, keepdims=True) + 1e-6
        num = jax.lax.dot_general(qf, kv, (((1,), (0,)), ((), ())),
                                  preferred_element_type=jnp.float32)
        o_l = num / denom
        o_l = jax.lax.dot_general(o_l, w, (((1,), (1,)), ((), ())),
                                  preferred_element_type=jnp.float32) + bb
        o_ref[0, g * BLKQ:(g + 1) * BLKQ, :] = o_l + o_s


def kernel(q, k, v, W_proj, b_proj):
    b, l, h, d = q.shape
    bh = b * h
    mb = l // BLKQ
    nb = l // BLKK
    topk = min(nb, int(TOPK_FRAC * nb))
    scale = 1.0 / math.sqrt(d)

    qt = jnp.transpose(q, (0, 2, 1, 3)).reshape(bh, l, d)
    kt = jnp.transpose(k, (0, 2, 1, 3)).reshape(bh, l, d)
    vt = jnp.transpose(v, (0, 2, 1, 3)).reshape(bh, l, d)

    km, kvsum, ksum, lut = pl.pallas_call(
        functools.partial(_stats_kernel, mb=mb, nb=nb, topk=topk),
        grid=(bh,),
        in_specs=[
            pl.BlockSpec((1, l, d), lambda i: (i, 0, 0)),
            pl.BlockSpec((1, l, d), lambda i: (i, 0, 0)),
            pl.BlockSpec((1, l, d), lambda i: (i, 0, 0)),
        ],
        out_specs=[
            pl.BlockSpec((1, 1, d), lambda i: (i, 0, 0)),
            pl.BlockSpec((1, d, d), lambda i: (i, 0, 0)),
            pl.BlockSpec((1, 1, d), lambda i: (i, 0, 0)),
            pl.BlockSpec((1, mb, topk), lambda i: (i, 0, 0)),
        ],
        out_shape=[
            jax.ShapeDtypeStruct((bh, 1, d), jnp.float32),
            jax.ShapeDtypeStruct((bh, d, d), jnp.float32),
            jax.ShapeDtypeStruct((bh, 1, d), jnp.float32),
            jax.ShapeDtypeStruct((bh, mb, topk), jnp.int32),
        ],
        compiler_params=pltpu.CompilerParams(
            dimension_semantics=("arbitrary",)),
    )(qt, kt, vt)

    mg = 16
    out = pl.pallas_call(
        functools.partial(_attn_kernel, topk=topk, scale=scale, mg=mg),
        grid=(bh, mb // mg),
        in_specs=[
            pl.BlockSpec(memory_space=pltpu.SMEM),
            pl.BlockSpec((1, mg * BLKQ, d), lambda i, j: (i, j, 0)),
            pl.BlockSpec((1, l, d), lambda i, j: (i, 0, 0)),
            pl.BlockSpec((1, l, d), lambda i, j: (i, 0, 0)),
            pl.BlockSpec((1, d, d), lambda i, j: (i, 0, 0)),
            pl.BlockSpec((1, 1, d), lambda i, j: (i, 0, 0)),
            pl.BlockSpec((d, d), lambda i, j: (0, 0)),
            pl.BlockSpec((1, d), lambda i, j: (0, 0)),
        ],
        out_specs=pl.BlockSpec((1, mg * BLKQ, d), lambda i, j: (i, j, 0)),
        out_shape=jax.ShapeDtypeStruct((bh, l, d), jnp.float32),
        scratch_shapes=[
            pltpu.VMEM((mg, topk * BLKK, d), jnp.bfloat16),
            pltpu.VMEM((mg, topk * BLKK, d), jnp.bfloat16),
        ],
        compiler_params=pltpu.CompilerParams(
            dimension_semantics=("arbitrary", "arbitrary")),
    )(lut, qt, kt, vt, kvsum, ksum, W_proj, b_proj.reshape(1, d))

    return jnp.transpose(out.reshape(b, h, l, d), (0, 2, 1, 3))


# speedup vs baseline: 1.9824x; 1.9824x over previous
import functools
import math

import jax
import jax.numpy as jnp
from jax.experimental import pallas as pl
from jax.experimental.pallas import tpu as pltpu

BLKQ, BLKK = 64, 32
TOPK_FRAC = 0.1


def _stats_kernel(q_ref, k_ref, v_ref, km_ref, kvsum_ref, ksum_ref, lut_ref,
                  *, mb, nb, topk):
    q = q_ref[0]
    k = k_ref[0]
    v = v_ref[0]
    l, d = q.shape
    km = jnp.mean(k, axis=0, keepdims=True)
    km_ref[0] = km
    pq = jnp.mean(q.reshape(mb, BLKQ, d), axis=1)
    pk = jnp.mean(k.reshape(nb, BLKK, d), axis=1) - km
    ps = jax.lax.dot_general(pq, pk, (((1,), (1,)), ((), ())),
                             precision=jax.lax.Precision.HIGHEST,
                             preferred_element_type=jnp.float32)
    col = jax.lax.broadcasted_iota(jnp.int32, ps.shape, 1)
    for t in range(topk):
        mx = jnp.max(ps, axis=1, keepdims=True)
        idx = jnp.min(jnp.where(ps >= mx, col, nb), axis=1, keepdims=True)
        lut_ref[0, :, t:t + 1] = idx
        ps = jnp.where(col == idx, -jnp.inf, ps)
    kf = jax.nn.softmax(k, axis=-1)
    kvsum_ref[0] = jax.lax.dot_general(kf, v, (((0,), (0,)), ((), ())),
                                       preferred_element_type=jnp.float32)
    ksum_ref[0] = jnp.sum(kf, axis=0, keepdims=True)


def _attn_kernel(lut_ref, q_ref, k_ref, v_ref, kvsum_ref, ksum_ref,
                 w_ref, b_ref, o_ref, kc_scr, vc_scr, s_scr, p_scr, os_scr,
                 *, topk, scale, mg):
    h = pl.program_id(0)
    jg = pl.program_id(1)
    for g in range(mg):
        m = jg * mg + g
        for t in range(topk):
            idx = lut_ref[h, m, t]
            off = idx * BLKK
            kc_scr[g, t * BLKK:(t + 1) * BLKK, :] = (
                k_ref[0, pl.ds(off, BLKK), :].astype(jnp.bfloat16))
            vc_scr[g, t * BLKK:(t + 1) * BLKK, :] = (
                v_ref[0, pl.ds(off, BLKK), :].astype(jnp.bfloat16))
    qall = q_ref[0]
    qs = (qall * scale).astype(jnp.bfloat16)
    for g in range(mg):
        s_scr[g] = jax.lax.dot_general(
            qs[g * BLKQ:(g + 1) * BLKQ, :], kc_scr[g],
            (((1,), (1,)), ((), ())), preferred_element_type=jnp.float32)
    sa = s_scr[...].reshape(mg * BLKQ, topk * BLKK)
    e = jnp.exp(sa - jnp.max(sa, axis=1, keepdims=True))
    pn = e / jnp.sum(e, axis=1, keepdims=True)
    p_scr[...] = pn.astype(jnp.bfloat16).reshape(mg, BLKQ, topk * BLKK)
    for g in range(mg):
        os_scr[g * BLKQ:(g + 1) * BLKQ, :] = jax.lax.dot_general(
            p_scr[g], vc_scr[g], (((1,), (0,)), ((), ())),
            preferred_element_type=jnp.float32)
    qf = jax.nn.softmax(qall, axis=-1)
    denom = jnp.sum(qf * ksum_ref[0], axis=1, keepdims=True) + 1e-6
    num = jax.lax.dot_general(qf, kvsum_ref[0], (((1,), (0,)), ((), ())),
                              preferred_element_type=jnp.float32)
    o_l = num / denom
    o_l = jax.lax.dot_general(o_l, w_ref[...], (((1,), (1,)), ((), ())),
                              preferred_element_type=jnp.float32) + b_ref[0]
    o_ref[0] = o_l + os_scr[...]


def kernel(q, k, v, W_proj, b_proj):
    b, l, h, d = q.shape
    bh = b * h
    mb = l // BLKQ
    nb = l // BLKK
    topk = min(nb, int(TOPK_FRAC * nb))
    scale = 1.0 / math.sqrt(d)

    qt = jnp.transpose(q, (0, 2, 1, 3)).reshape(bh, l, d)
    kt = jnp.transpose(k, (0, 2, 1, 3)).reshape(bh, l, d)
    vt = jnp.transpose(v, (0, 2, 1, 3)).reshape(bh, l, d)

    km, kvsum, ksum, lut = pl.pallas_call(
        functools.partial(_stats_kernel, mb=mb, nb=nb, topk=topk),
        grid=(bh,),
        in_specs=[
            pl.BlockSpec((1, l, d), lambda i: (i, 0, 0)),
            pl.BlockSpec((1, l, d), lambda i: (i, 0, 0)),
            pl.BlockSpec((1, l, d), lambda i: (i, 0, 0)),
        ],
        out_specs=[
            pl.BlockSpec((1, 1, d), lambda i: (i, 0, 0)),
            pl.BlockSpec((1, d, d), lambda i: (i, 0, 0)),
            pl.BlockSpec((1, 1, d), lambda i: (i, 0, 0)),
            pl.BlockSpec((1, mb, topk), lambda i: (i, 0, 0)),
        ],
        out_shape=[
            jax.ShapeDtypeStruct((bh, 1, d), jnp.float32),
            jax.ShapeDtypeStruct((bh, d, d), jnp.float32),
            jax.ShapeDtypeStruct((bh, 1, d), jnp.float32),
            jax.ShapeDtypeStruct((bh, mb, topk), jnp.int32),
        ],
        compiler_params=pltpu.CompilerParams(
            dimension_semantics=("arbitrary",)),
    )(qt, kt, vt)

    mg = 16
    out = pl.pallas_call(
        functools.partial(_attn_kernel, topk=topk, scale=scale, mg=mg),
        grid=(bh, mb // mg),
        in_specs=[
            pl.BlockSpec(memory_space=pltpu.SMEM),
            pl.BlockSpec((1, mg * BLKQ, d), lambda i, j: (i, j, 0)),
            pl.BlockSpec((1, l, d), lambda i, j: (i, 0, 0)),
            pl.BlockSpec((1, l, d), lambda i, j: (i, 0, 0)),
            pl.BlockSpec((1, d, d), lambda i, j: (i, 0, 0)),
            pl.BlockSpec((1, 1, d), lambda i, j: (i, 0, 0)),
            pl.BlockSpec((d, d), lambda i, j: (0, 0)),
            pl.BlockSpec((1, d), lambda i, j: (0, 0)),
        ],
        out_specs=pl.BlockSpec((1, mg * BLKQ, d), lambda i, j: (i, j, 0)),
        out_shape=jax.ShapeDtypeStruct((bh, l, d), jnp.float32),
        scratch_shapes=[
            pltpu.VMEM((mg, topk * BLKK, d), jnp.bfloat16),
            pltpu.VMEM((mg, topk * BLKK, d), jnp.bfloat16),
            pltpu.VMEM((mg, BLKQ, topk * BLKK), jnp.float32),
            pltpu.VMEM((mg, BLKQ, topk * BLKK), jnp.bfloat16),
            pltpu.VMEM((mg * BLKQ, d), jnp.float32),
        ],
        compiler_params=pltpu.CompilerParams(
            dimension_semantics=("arbitrary", "arbitrary")),
    )(lut, qt, kt, vt, kvsum, ksum, W_proj, b_proj.reshape(1, d))

    return jnp.transpose(out.reshape(b, h, l, d), (0, 2, 1, 3))


# speedup vs baseline: 2.1644x; 1.0918x over previous
import functools
import math

import jax
import jax.numpy as jnp
from jax import lax
from jax.experimental import pallas as pl
from jax.experimental.pallas import tpu as pltpu
from jax.experimental.pallas import tpu_sc as plsc

BLKQ, BLKK = 64, 32
TOPK_FRAC = 0.1
LUTPAD = 16


def _stats_kernel(q_ref, k_ref, v_ref, km_ref, kvsum_ref, ksum_ref, ps_ref,
                  *, mb, nb, topk):
    q = q_ref[0]
    k = k_ref[0]
    v = v_ref[0]
    l, d = q.shape
    km = jnp.mean(k, axis=0, keepdims=True)
    km_ref[0] = km
    pq = jnp.mean(q.reshape(mb, BLKQ, d), axis=1)
    pk = jnp.mean(k.reshape(nb, BLKK, d), axis=1) - km
    ps_ref[0] = jax.lax.dot_general(pq, pk, (((1,), (1,)), ((), ())),
                                    precision=jax.lax.Precision.HIGHEST,
                                    preferred_element_type=jnp.float32)
    kf = jax.nn.softmax(k, axis=-1)
    kvsum_ref[0] = jax.lax.dot_general(kf, v, (((0,), (0,)), ((), ())),
                                       preferred_element_type=jnp.float32)
    ksum_ref[0] = jnp.sum(kf, axis=0, keepdims=True)


def _sc_topk(ps_hbm, lut_hbm, ps_v, lut_v, *, rows_per, nb, topk):
    info = plsc.get_sparse_core_info()
    nc = info.num_cores
    wid = lax.axis_index("s") * nc + lax.axis_index("c")
    base = wid * rows_per
    pltpu.sync_copy(ps_hbm.at[pl.ds(base, rows_per)], ps_v)
    nvec = nb // 16
    lanes = jnp.arange(16, dtype=jnp.int32)
    perms = [jnp.bitwise_xor(lanes, sh) for sh in (1, 2, 4, 8)]

    gdn = lax.GatherDimensionNumbers(offset_dims=(), collapsed_slice_dims=(0,),
                                     start_index_map=(0,))

    def _shuf(x, p):
        return lax.gather(x, p[:, None], gdn, (1,),
                          mode=lax.GatherScatterMode.PROMISE_IN_BOUNDS)

    def row_body(r, _):
        vecs = [ps_v[r, pl.ds(j * 16, 16)] for j in range(nvec)]
        out = jnp.zeros((16,), jnp.int32)
        for t in range(topk):
            m = vecs[0]
            for j in range(1, nvec):
                m = jnp.maximum(m, vecs[j])
            for p in perms:
                m = jnp.maximum(m, _shuf(m, p))
            idx = jnp.full((16,), nb, jnp.int32)
            for j in range(nvec):
                idx = jnp.minimum(idx, jnp.where(vecs[j] >= m,
                                                 lanes + 16 * j, nb))
            for p in perms:
                idx = jnp.minimum(idx, _shuf(idx, p))
            out = jnp.where(lanes == t, idx, out)
            for j in range(nvec):
                vecs[j] = jnp.where(lanes + 16 * j == idx,
                                    jnp.float32(-3.0e38), vecs[j])
        lut_v[r] = out
        return _

    lax.fori_loop(0, rows_per, row_body, 0)
    pltpu.sync_copy(lut_v, lut_hbm.at[pl.ds(base, rows_per)])


def _topk_lut(ps, nb, topk):
    rows = ps.shape[0] * ps.shape[1]
    info = plsc.get_sparse_core_info()
    nw = info.num_cores * info.num_subcores
    rows_per = rows // nw
    mesh = plsc.VectorSubcoreMesh(core_axis_name="c", subcore_axis_name="s")
    f = functools.partial(
        pl.kernel,
        mesh=mesh,
        out_type=jax.ShapeDtypeStruct((rows, LUTPAD), jnp.int32),
        scratch_types=[
            pltpu.VMEM((rows_per, nb), jnp.float32),
            pltpu.VMEM((rows_per, LUTPAD), jnp.int32),
        ],
    )(functools.partial(_sc_topk, rows_per=rows_per, nb=nb, topk=topk))
    return f(ps.reshape(rows, nb)).reshape(ps.shape[0], ps.shape[1], LUTPAD)


def _attn_kernel(lut_ref, q_ref, k_ref, v_ref, kvsum_ref, ksum_ref,
                 w_ref, b_ref, o_ref, kc_scr, vc_scr, s_scr, p_scr, os_scr,
                 *, topk, scale, mg):
    h = pl.program_id(0)
    jg = pl.program_id(1)
    for g in range(mg):
        m = jg * mg + g
        for t in range(topk):
            idx = lut_ref[h, m, t]
            off = idx * BLKK
            kc_scr[g, t * BLKK:(t + 1) * BLKK, :] = (
                k_ref[0, pl.ds(off, BLKK), :].astype(jnp.bfloat16))
            vc_scr[g, t * BLKK:(t + 1) * BLKK, :] = (
                v_ref[0, pl.ds(off, BLKK), :].astype(jnp.bfloat16))
    qall = q_ref[0]
    qs = (qall * scale).astype(jnp.bfloat16)
    for g in range(mg):
        s_scr[g] = jax.lax.dot_general(
            qs[g * BLKQ:(g + 1) * BLKQ, :], kc_scr[g],
            (((1,), (1,)), ((), ())), preferred_element_type=jnp.float32)
    sa = s_scr[...].reshape(mg * BLKQ, topk * BLKK)
    e = jnp.exp(sa - jnp.max(sa, axis=1, keepdims=True))
    pn = e / jnp.sum(e, axis=1, keepdims=True)
    p_scr[...] = pn.astype(jnp.bfloat16).reshape(mg, BLKQ, topk * BLKK)
    for g in range(mg):
        os_scr[g * BLKQ:(g + 1) * BLKQ, :] = jax.lax.dot_general(
            p_scr[g], vc_scr[g], (((1,), (0,)), ((), ())),
            preferred_element_type=jnp.float32)
    qf = jax.nn.softmax(qall, axis=-1)
    denom = jnp.sum(qf * ksum_ref[0], axis=1, keepdims=True) + 1e-6
    num = jax.lax.dot_general(qf, kvsum_ref[0], (((1,), (0,)), ((), ())),
                              preferred_element_type=jnp.float32)
    o_l = num / denom
    o_l = jax.lax.dot_general(o_l, w_ref[...], (((1,), (1,)), ((), ())),
                              preferred_element_type=jnp.float32) + b_ref[0]
    o_ref[0] = o_l + os_scr[...]


def kernel(q, k, v, W_proj, b_proj):
    b, l, h, d = q.shape
    bh = b * h
    mb = l // BLKQ
    nb = l // BLKK
    topk = min(nb, int(TOPK_FRAC * nb))
    scale = 1.0 / math.sqrt(d)

    qt = jnp.transpose(q, (0, 2, 1, 3)).reshape(bh, l, d)
    kt = jnp.transpose(k, (0, 2, 1, 3)).reshape(bh, l, d)
    vt = jnp.transpose(v, (0, 2, 1, 3)).reshape(bh, l, d)

    km, kvsum, ksum, ps = pl.pallas_call(
        functools.partial(_stats_kernel, mb=mb, nb=nb, topk=topk),
        grid=(bh,),
        in_specs=[
            pl.BlockSpec((1, l, d), lambda i: (i, 0, 0)),
            pl.BlockSpec((1, l, d), lambda i: (i, 0, 0)),
            pl.BlockSpec((1, l, d), lambda i: (i, 0, 0)),
        ],
        out_specs=[
            pl.BlockSpec((1, 1, d), lambda i: (i, 0, 0)),
            pl.BlockSpec((1, d, d), lambda i: (i, 0, 0)),
            pl.BlockSpec((1, 1, d), lambda i: (i, 0, 0)),
            pl.BlockSpec((1, mb, nb), lambda i: (i, 0, 0)),
        ],
        out_shape=[
            jax.ShapeDtypeStruct((bh, 1, d), jnp.float32),
            jax.ShapeDtypeStruct((bh, d, d), jnp.float32),
            jax.ShapeDtypeStruct((bh, 1, d), jnp.float32),
            jax.ShapeDtypeStruct((bh, mb, nb), jnp.float32),
        ],
        compiler_params=pltpu.CompilerParams(
            dimension_semantics=("arbitrary",)),
    )(qt, kt, vt)

    lut = _topk_lut(ps, nb, topk)

    mg = 16
    out = pl.pallas_call(
        functools.partial(_attn_kernel, topk=topk, scale=scale, mg=mg),
        grid=(bh, mb // mg),
        in_specs=[
            pl.BlockSpec(memory_space=pltpu.SMEM),
            pl.BlockSpec((1, mg * BLKQ, d), lambda i, j: (i, j, 0)),
            pl.BlockSpec((1, l, d), lambda i, j: (i, 0, 0)),
            pl.BlockSpec((1, l, d), lambda i, j: (i, 0, 0)),
            pl.BlockSpec((1, d, d), lambda i, j: (i, 0, 0)),
            pl.BlockSpec((1, 1, d), lambda i, j: (i, 0, 0)),
            pl.BlockSpec((d, d), lambda i, j: (0, 0)),
            pl.BlockSpec((1, d), lambda i, j: (0, 0)),
        ],
        out_specs=pl.BlockSpec((1, mg * BLKQ, d), lambda i, j: (i, j, 0)),
        out_shape=jax.ShapeDtypeStruct((bh, l, d), jnp.float32),
        scratch_shapes=[
            pltpu.VMEM((mg, topk * BLKK, d), jnp.bfloat16),
            pltpu.VMEM((mg, topk * BLKK, d), jnp.bfloat16),
            pltpu.VMEM((mg, BLKQ, topk * BLKK), jnp.float32),
            pltpu.VMEM((mg, BLKQ, topk * BLKK), jnp.bfloat16),
            pltpu.VMEM((mg * BLKQ, d), jnp.float32),
        ],
        compiler_params=pltpu.CompilerParams(
            dimension_semantics=("arbitrary", "arbitrary")),
    )(lut, qt, kt, vt, kvsum, ksum, W_proj, b_proj.reshape(1, d))

    return jnp.transpose(out.reshape(b, h, l, d), (0, 2, 1, 3))
